# baseline (speedup 1.0000x reference)
import functools

import jax
import jax.numpy as jnp
from jax import lax
from jax.experimental import pallas as pl
from jax.experimental.pallas import tpu as pltpu
from jax.experimental.pallas import tpu_sc as plsc

B = 8192
T = 4
NEIGH = 16
E = 128
NW = 32
NPW = B // NW
CHUNK_NODES = 2
ROWS_PER_CHUNK = CHUNK_NODES * T * NEIGH
SEGS_PER_CHUNK = CHUNK_NODES * T
NCHUNKS = NPW // CHUNK_NODES
NBUF = 2
NSTEPS = NCHUNKS // NBUF
IDX_PER_WORKER = NPW * T * NEIGH


def _sc_gather_sum(table, nemb, neigh_flat, inp):
  mesh = plsc.VectorSubcoreMesh(core_axis_name="c", subcore_axis_name="s")

  @functools.partial(
      pl.kernel,
      out_type=(
          jax.ShapeDtypeStruct((B, T * E), jnp.float32),
          jax.ShapeDtypeStruct((B, E), jnp.float32),
      ),
      mesh=mesh,
      scratch_types=(
          pltpu.VMEM((IDX_PER_WORKER,), jnp.int32),
          [pltpu.VMEM((ROWS_PER_CHUNK,), jnp.int32) for _ in range(NBUF)],
          [pltpu.VMEM((ROWS_PER_CHUNK, E), jnp.float32) for _ in range(NBUF)],
          [pltpu.VMEM((CHUNK_NODES, T * E), jnp.float32) for _ in range(NBUF)],
          pltpu.VMEM((NPW, E), jnp.float32),
          pltpu.VMEM((NPW,), jnp.int32),
          [pltpu.SemaphoreType.DMA for _ in range(NBUF)],
          [pltpu.SemaphoreType.DMA for _ in range(NBUF)],
          pltpu.SemaphoreType.DMA,
      ),
  )
  def k(table_hbm, nemb_hbm, nidx_hbm, inp_hbm, nsum_hbm, nrows_hbm,
        idx_raw, fidxs, gbufs, obufs, nbuf_rows, in_idx, gsems, osems, nsem):
    wid = lax.axis_index("s") * 2 + lax.axis_index("c")
    base_n = wid * NPW

    pltpu.sync_copy(inp_hbm.at[pl.ds(base_n, NPW)], in_idx)
    cp_n0 = pltpu.async_copy(
        nemb_hbm.at[in_idx.at[pl.ds(0, 128)]], nbuf_rows.at[pl.ds(0, 128)],
        nsem)
    cp_n1 = pltpu.async_copy(
        nemb_hbm.at[in_idx.at[pl.ds(128, 128)]], nbuf_rows.at[pl.ds(128, 128)],
        nsem)

    pltpu.sync_copy(nidx_hbm.at[pl.ds(base_n * T * NEIGH, IDX_PER_WORKER)],
                    idx_raw)

    def issue(g, b):
      for j in range(ROWS_PER_CHUNK // 16):
        r = idx_raw[pl.ds(g * ROWS_PER_CHUNK + j * 16, 16)]
        fidxs[b][pl.ds(j * 16, 16)] = r * T + (j % T)
      pltpu.async_copy(table_hbm.at[fidxs[b]], gbufs[b], gsems[b])

    for b in range(NBUF):
      issue(b, b)

    @pl.loop(0, NSTEPS)
    def _(step):
      for b in range(NBUF):
        g = step * NBUF + b
        pltpu.make_async_copy(table_hbm.at[fidxs[b]], gbufs[b],
                              gsems[b]).wait()

        @pl.when(step >= 1)
        def _():
          pltpu.make_async_copy(
              obufs[b], nsum_hbm.at[pl.ds(base_n + CHUNK_NODES * g,
                                          CHUNK_NODES)], osems[b]).wait()

        for seg in range(SEGS_PER_CHUNK):
          node, typ = seg // T, seg % T
          for cc in range(E // 16):
            vs = [gbufs[b][seg * NEIGH + r, pl.ds(cc * 16, 16)]
                  for r in range(NEIGH)]
            while len(vs) > 1:
              vs = [vs[2 * i] + vs[2 * i + 1] for i in range(len(vs) // 2)]
            obufs[b][node, pl.ds(typ * E + cc * 16, 16)] = vs[0]

        pltpu.async_copy(
            obufs[b], nsum_hbm.at[pl.ds(base_n + CHUNK_NODES * g,
                                        CHUNK_NODES)], osems[b])

        @pl.when(step <= NSTEPS - 2)
        def _():
          issue(g + NBUF, b)

    for b in range(NBUF):
      pltpu.make_async_copy(obufs[b], nsum_hbm.at[pl.ds(0, CHUNK_NODES)],
                            osems[b]).wait()
    cp_n0.wait()
    cp_n1.wait()
    pltpu.sync_copy(nbuf_rows, nrows_hbm.at[pl.ds(base_n, NPW)])

  return k(table, nemb, neigh_flat, inp)


def _tc_body(x_ref, ne_ref, oh_ref, s1_ref, s2_ref, w_ref, o_ref):
  X = x_ref[...]
  oh = oh_ref[...]
  S1 = s1_ref[...]
  S2 = s2_ref[...]
  W = w_ref[...]
  Xts = [X[:, E * t:E * (t + 1)] for t in range(T)]
  ls = []
  for t in range(T):
    H = jnp.tanh(jnp.dot(Xts[t], S1, preferred_element_type=jnp.float32))
    Lk = jnp.dot(H, S2, preferred_element_type=jnp.float32)
    ls.append(jnp.sum(Lk * oh, axis=1, keepdims=True))
  m = jnp.maximum(jnp.maximum(ls[0], ls[1]), jnp.maximum(ls[2], ls[3]))
  es = [jnp.exp(l - m) for l in ls]
  denom = es[0] + es[1] + es[2] + es[3]
  comb = (es[0] * Xts[0] + es[1] * Xts[1] + es[2] * Xts[2]
          + es[3] * Xts[3]) / denom
  Y = jnp.dot(comb, W, preferred_element_type=jnp.float32)
  sel = (oh[:, 0:1] * Y[:, 0:E] + oh[:, 1:2] * Y[:, E:2 * E]
         + oh[:, 2:3] * Y[:, 2 * E:3 * E] + oh[:, 3:4] * Y[:, 3 * E:4 * E])
  out = ne_ref[...] + sel
  nrm = jnp.sqrt(jnp.sum(out * out, axis=1, keepdims=True))
  o_ref[...] = out / jnp.maximum(nrm, 1e-12)


def _tc_combine(x2d, nrows, oh, S1, S2, W):
  BB = 1024
  grid = (B // BB,)
  return pl.pallas_call(
      _tc_body,
      grid=grid,
      in_specs=[
          pl.BlockSpec((BB, T * E), lambda i: (i, 0)),
          pl.BlockSpec((BB, E), lambda i: (i, 0)),
          pl.BlockSpec((BB, 128), lambda i: (i, 0)),
          pl.BlockSpec((E, 128), lambda i: (0, 0)),
          pl.BlockSpec((128, 128), lambda i: (0, 0)),
          pl.BlockSpec((E, T * E), lambda i: (0, 0)),
      ],
      out_specs=pl.BlockSpec((BB, E), lambda i: (i, 0)),
      out_shape=jax.ShapeDtypeStruct((B, E), jnp.float32),
  )(x2d, nrows, oh, S1, S2, W)


def kernel(inputs, node_types, node_neigh, node_embeddings,
           node_type_embeddings, trans_weights, trans_weights_s1,
           trans_weights_s2):
  flat_table = node_type_embeddings.reshape(-1, E)
  neigh_flat = node_neigh.reshape(-1).astype(jnp.int32)
  inp = inputs.astype(jnp.int32)

  nsum, nrows = _sc_gather_sum(flat_table, node_embeddings, neigh_flat, inp)

  S1 = jnp.transpose(trans_weights_s1, (1, 0, 2)).reshape(E, T * 32)
  S2 = jnp.zeros((T * 32, 128), jnp.float32)
  for kk in range(T):
    S2 = S2.at[32 * kk:32 * (kk + 1), kk].set(trans_weights_s2[kk, :, 0])
  W = jnp.transpose(trans_weights, (1, 0, 2)).reshape(E, T * E)
  oh = (node_types[:, None] == jnp.arange(T)[None, :]).astype(jnp.float32)
  oh = jnp.pad(oh, ((0, 0), (0, 128 - T)))

  return _tc_combine(nsum, nrows, oh, S1, S2, W)


# speedup vs baseline: 2.2494x; 2.2494x over previous
import functools

import jax
import jax.numpy as jnp
from jax import lax
from jax.experimental import pallas as pl
from jax.experimental.pallas import tpu as pltpu
from jax.experimental.pallas import tpu_sc as plsc

B = 8192
T = 4
NEIGH = 16
E = 128
NW = 32
NPW = B // NW
CHUNK_NODES = 2
ROWS_PER_CHUNK = CHUNK_NODES * T * NEIGH
SEGS_PER_CHUNK = CHUNK_NODES * T
NCHUNKS = NPW // CHUNK_NODES
NBUF = 4
NSTEPS = NCHUNKS // NBUF
IDX_PER_WORKER = NPW * T * NEIGH


def _sc_gather_sum(table, nemb, neigh_flat, inp):
  mesh = plsc.VectorSubcoreMesh(core_axis_name="c", subcore_axis_name="s")

  @functools.partial(
      pl.kernel,
      out_type=(
          jax.ShapeDtypeStruct((B, T * E), jnp.float32),
          jax.ShapeDtypeStruct((B, E), jnp.float32),
      ),
      mesh=mesh,
      scratch_types=(
          pltpu.VMEM((IDX_PER_WORKER,), jnp.int32),
          [pltpu.VMEM((ROWS_PER_CHUNK,), jnp.int32) for _ in range(NBUF)],
          [pltpu.VMEM((ROWS_PER_CHUNK, E), jnp.float32) for _ in range(NBUF)],
          [pltpu.VMEM((CHUNK_NODES, T * E), jnp.float32) for _ in range(NBUF)],
          pltpu.VMEM((NPW, E), jnp.float32),
          pltpu.VMEM((NPW,), jnp.int32),
          [pltpu.SemaphoreType.DMA for _ in range(NBUF)],
          [pltpu.SemaphoreType.DMA for _ in range(NBUF)],
          pltpu.SemaphoreType.DMA,
      ),
  )
  def k(table_hbm, nemb_hbm, nidx_hbm, inp_hbm, nsum_hbm, nrows_hbm,
        idx_raw, fidxs, gbufs, obufs, nbuf_rows, in_idx, gsems, osems, nsem):
    wid = lax.axis_index("s") * 2 + lax.axis_index("c")
    base_n = wid * NPW

    pltpu.sync_copy(inp_hbm.at[pl.ds(base_n, NPW)], in_idx)
    cp_n0 = pltpu.async_copy(
        nemb_hbm.at[in_idx.at[pl.ds(0, 128)]], nbuf_rows.at[pl.ds(0, 128)],
        nsem)
    cp_n1 = pltpu.async_copy(
        nemb_hbm.at[in_idx.at[pl.ds(128, 128)]], nbuf_rows.at[pl.ds(128, 128)],
        nsem)

    pltpu.sync_copy(nidx_hbm.at[pl.ds(base_n * T * NEIGH, IDX_PER_WORKER)],
                    idx_raw)

    def issue(g, b):
      for j in range(ROWS_PER_CHUNK // 16):
        r = idx_raw[pl.ds(g * ROWS_PER_CHUNK + j * 16, 16)]
        fidxs[b][pl.ds(j * 16, 16)] = r * T + (j % T)
      pltpu.async_copy(table_hbm.at[fidxs[b]], gbufs[b], gsems[b])

    for b in range(NBUF):
      issue(b, b)

    @pl.loop(0, NSTEPS)
    def _(step):
      for b in range(NBUF):
        g = step * NBUF + b
        pltpu.make_async_copy(table_hbm.at[fidxs[b]], gbufs[b],
                              gsems[b]).wait()

        @pl.when(step >= 1)
        def _():
          pltpu.make_async_copy(
              obufs[b], nsum_hbm.at[pl.ds(base_n + CHUNK_NODES * g,
                                          CHUNK_NODES)], osems[b]).wait()

        @plsc.parallel_loop(0, SEGS_PER_CHUNK, unroll=2)
        def _(seg):
          node = seg // T
          typ = seg % T
          for cc in range(E // 16):
            vs = [gbufs[b][seg * NEIGH + r, pl.ds(cc * 16, 16)]
                  for r in range(NEIGH)]
            while len(vs) > 1:
              vs = [vs[2 * i] + vs[2 * i + 1] for i in range(len(vs) // 2)]
            obufs[b][node, pl.ds(typ * E + cc * 16, 16)] = vs[0]

        pltpu.async_copy(
            obufs[b], nsum_hbm.at[pl.ds(base_n + CHUNK_NODES * g,
                                        CHUNK_NODES)], osems[b])

        @pl.when(step <= NSTEPS - 2)
        def _():
          issue(g + NBUF, b)

    for b in range(NBUF):
      pltpu.make_async_copy(obufs[b], nsum_hbm.at[pl.ds(0, CHUNK_NODES)],
                            osems[b]).wait()
    cp_n0.wait()
    cp_n1.wait()
    pltpu.sync_copy(nbuf_rows, nrows_hbm.at[pl.ds(base_n, NPW)])

  return k(table, nemb, neigh_flat, inp)


def _tc_body(x_ref, ne_ref, oh_ref, s1_ref, s2_ref, w_ref, o_ref):
  X = x_ref[...]
  oh = oh_ref[...]
  S1 = s1_ref[...]
  S2 = s2_ref[...]
  W = w_ref[...]
  Xts = [X[:, E * t:E * (t + 1)] for t in range(T)]
  ls = []
  for t in range(T):
    H = jnp.tanh(jnp.dot(Xts[t], S1, preferred_element_type=jnp.float32))
    Lk = jnp.dot(H, S2, preferred_element_type=jnp.float32)
    ls.append(jnp.sum(Lk * oh, axis=1, keepdims=True))
  m = jnp.maximum(jnp.maximum(ls[0], ls[1]), jnp.maximum(ls[2], ls[3]))
  es = [jnp.exp(l - m) for l in ls]
  denom = es[0] + es[1] + es[2] + es[3]
  comb = (es[0] * Xts[0] + es[1] * Xts[1] + es[2] * Xts[2]
          + es[3] * Xts[3]) / denom
  Y = jnp.dot(comb, W, preferred_element_type=jnp.float32)
  sel = (oh[:, 0:1] * Y[:, 0:E] + oh[:, 1:2] * Y[:, E:2 * E]
         + oh[:, 2:3] * Y[:, 2 * E:3 * E] + oh[:, 3:4] * Y[:, 3 * E:4 * E])
  out = ne_ref[...] + sel
  nrm = jnp.sqrt(jnp.sum(out * out, axis=1, keepdims=True))
  o_ref[...] = out / jnp.maximum(nrm, 1e-12)


def _tc_combine(x2d, nrows, oh, S1, S2, W):
  BB = 1024
  grid = (B // BB,)
  return pl.pallas_call(
      _tc_body,
      grid=grid,
      in_specs=[
          pl.BlockSpec((BB, T * E), lambda i: (i, 0)),
          pl.BlockSpec((BB, E), lambda i: (i, 0)),
          pl.BlockSpec((BB, 128), lambda i: (i, 0)),
          pl.BlockSpec((E, 128), lambda i: (0, 0)),
          pl.BlockSpec((128, 128), lambda i: (0, 0)),
          pl.BlockSpec((E, T * E), lambda i: (0, 0)),
      ],
      out_specs=pl.BlockSpec((BB, E), lambda i: (i, 0)),
      out_shape=jax.ShapeDtypeStruct((B, E), jnp.float32),
  )(x2d, nrows, oh, S1, S2, W)


def kernel(inputs, node_types, node_neigh, node_embeddings,
           node_type_embeddings, trans_weights, trans_weights_s1,
           trans_weights_s2):
  flat_table = node_type_embeddings.reshape(-1, E)
  neigh_flat = node_neigh.reshape(-1).astype(jnp.int32)
  inp = inputs.astype(jnp.int32)

  nsum, nrows = _sc_gather_sum(flat_table, node_embeddings, neigh_flat, inp)

  S1 = jnp.transpose(trans_weights_s1, (1, 0, 2)).reshape(E, T * 32)
  S2 = jnp.zeros((T * 32, 128), jnp.float32)
  for kk in range(T):
    S2 = S2.at[32 * kk:32 * (kk + 1), kk].set(trans_weights_s2[kk, :, 0])
  W = jnp.transpose(trans_weights, (1, 0, 2)).reshape(E, T * E)
  oh = (node_types[:, None] == jnp.arange(T)[None, :]).astype(jnp.float32)
  oh = jnp.pad(oh, ((0, 0), (0, 128 - T)))

  return _tc_combine(nsum, nrows, oh, S1, S2, W)
